# Initial kernel scaffold; baseline (speedup 1.0000x reference)
import functools

import jax
import jax.numpy as jnp
from jax.experimental import pallas as pl
from jax.experimental.pallas import tpu as pltpu

N_EDGES = 320000
EB = 1280
N_BLOCKS = N_EDGES // EB


def _mlp_body(agg_ref, w1_ref, b1_ref, w2_ref, b2_ref, w3_ref, b3_ref, out_ref):
    a = agg_ref[...]
    h = jnp.dot(a, w1_ref[...], preferred_element_type=jnp.float32) + b1_ref[...]
    h = jax.nn.gelu(h, approximate=False)
    h = jnp.dot(h, w2_ref[...], preferred_element_type=jnp.float32) + b2_ref[...]
    h = jax.nn.gelu(h, approximate=False)
    out_ref[...] = jnp.dot(h, w3_ref[...], preferred_element_type=jnp.float32) + b3_ref[...]


@jax.jit
def _mlp(agg8, W1p, b1, W2, b2, W3, b3):
    return pl.pallas_call(
        _mlp_body,
        grid=(N_BLOCKS,),
        in_specs=[
            pl.BlockSpec((EB, 8), lambda i: (i, 0)),
            pl.BlockSpec((8, 256), lambda i: (0, 0)),
            pl.BlockSpec((1, 256), lambda i: (0, 0)),
            pl.BlockSpec((256, 256), lambda i: (0, 0)),
            pl.BlockSpec((1, 256), lambda i: (0, 0)),
            pl.BlockSpec((256, 128), lambda i: (0, 0)),
            pl.BlockSpec((1, 128), lambda i: (0, 0)),
        ],
        out_specs=pl.BlockSpec((EB, 128), lambda i: (i, 0)),
        out_shape=jax.ShapeDtypeStruct((N_EDGES, 128), jnp.float32),
    )(agg8, W1p, b1, W2, b2, W3, b3)


def kernel(y_pos, x_pos, edge_index, f_y, W1, b1, W2, b2, W3, b3):
    qry = edge_index[0]
    src = edge_index[1]
    agg = jnp.concatenate([y_pos[src], x_pos[qry]], axis=1)
    agg8 = jnp.pad(agg, ((0, 0), (0, 2)))
    W1p = jnp.pad(W1, ((0, 2), (0, 0)))
    kern = _mlp(agg8, W1p, b1[None, :], W2, b2[None, :], W3, b3[None, :])
    rep = kern * f_y[src]
    n = x_pos.shape[0]
    sums = jax.ops.segment_sum(rep, qry, num_segments=n)
    counts = jax.ops.segment_sum(jnp.ones((N_EDGES,), jnp.float32), qry, num_segments=n)
    return sums / jnp.clip(counts, 1.0, None)[:, None]


# speedup vs baseline: 1.3437x; 1.3437x over previous
import functools

import jax
import jax.numpy as jnp
from jax.experimental import pallas as pl
from jax.experimental.pallas import tpu as pltpu

N_EDGES = 320000
EB = 1280
N_BLOCKS = N_EDGES // EB


def _gelu(x):
    return 0.5 * x * (1.0 + jax.lax.erf(x * 0.7071067811865476))


def _mlp_body(agg_ref, w1_ref, b1_ref, w2_ref, b2_ref, w3_ref, b3_ref, out_ref):
    a = agg_ref[...]
    h = jnp.dot(a, w1_ref[...], preferred_element_type=jnp.float32) + b1_ref[...]
    h = _gelu(h)
    h = jnp.dot(h, w2_ref[...], preferred_element_type=jnp.float32) + b2_ref[...]
    h = _gelu(h)
    out_ref[...] = jnp.dot(h, w3_ref[...], preferred_element_type=jnp.float32) + b3_ref[...]


@jax.jit
def _mlp(agg8, W1p, b1, W2, b2, W3, b3):
    return pl.pallas_call(
        _mlp_body,
        grid=(N_BLOCKS,),
        in_specs=[
            pl.BlockSpec((EB, 8), lambda i: (i, 0)),
            pl.BlockSpec((8, 256), lambda i: (0, 0)),
            pl.BlockSpec((1, 256), lambda i: (0, 0)),
            pl.BlockSpec((256, 256), lambda i: (0, 0)),
            pl.BlockSpec((1, 256), lambda i: (0, 0)),
            pl.BlockSpec((256, 128), lambda i: (0, 0)),
            pl.BlockSpec((1, 128), lambda i: (0, 0)),
        ],
        out_specs=pl.BlockSpec((EB, 128), lambda i: (i, 0)),
        out_shape=jax.ShapeDtypeStruct((N_EDGES, 128), jnp.float32),
    )(agg8, W1p, b1, W2, b2, W3, b3)


def kernel(y_pos, x_pos, edge_index, f_y, W1, b1, W2, b2, W3, b3):
    qry = edge_index[0]
    src = edge_index[1]
    agg = jnp.concatenate([y_pos[src], x_pos[qry]], axis=1)
    agg8 = jnp.pad(agg, ((0, 0), (0, 2)))
    W1p = jnp.pad(W1, ((0, 2), (0, 0)))
    kern = _mlp(agg8, W1p, b1[None, :], W2, b2[None, :], W3, b3[None, :])
    rep = kern * f_y[src]
    n = x_pos.shape[0]
    sums = jax.ops.segment_sum(rep, qry, num_segments=n)
    counts = jax.ops.segment_sum(jnp.ones((N_EDGES,), jnp.float32), qry, num_segments=n)
    return sums / jnp.clip(counts, 1.0, None)[:, None]


# speedup vs baseline: 5.2397x; 3.8995x over previous
import functools

import jax
import jax.numpy as jnp
from jax import lax
from jax.experimental import pallas as pl
from jax.experimental.pallas import tpu as pltpu
from jax.experimental.pallas import tpu_sc as plsc

E = 320000
N = 10000
F = 128
NC = 2
NS = 16
L = 16
NW = NC * NS
EPW = E // NW
CH = 80
NCH = EPW // CH
NTO = 10
RPT = N // NTO
ZR = 40
WR = 200

EB = 1280
N_BLOCKS = E // EB
NB = 1250


def _scA_body(y_hbm, x_hbm, src_hbm, qry_hbm, out_hbm, y_v, x_v, src_v, qry_v, o_v):
    wid = lax.axis_index("s") * NC + lax.axis_index("c")
    base = wid * EPW
    pltpu.sync_copy(y_hbm, y_v)
    pltpu.sync_copy(x_hbm, x_v)

    def zero(j, c):
        o_v[pl.ds(j * L, L)] = jnp.zeros((L,), jnp.float32)
        return c

    lax.fori_loop(0, CH * 8 // L, zero, 0)
    rows0 = lax.iota(jnp.int32, L)

    def chunk(i, carry):
        eb = base + i * CH
        pltpu.sync_copy(src_hbm.at[pl.ds(eb, CH)], src_v)
        pltpu.sync_copy(qry_hbm.at[pl.ds(eb, CH)], qry_v)
        for v in range(CH // L):
            s3 = src_v[pl.ds(v * L, L)] * 3
            q3 = qry_v[pl.ds(v * L, L)] * 3
            o8 = (rows0 + v * L) * 8
            for c in range(3):
                plsc.store_scatter(o_v, [o8 + c], plsc.load_gather(y_v, [s3 + c]))
                plsc.store_scatter(o_v, [o8 + 3 + c], plsc.load_gather(x_v, [q3 + c]))
        pltpu.sync_copy(o_v, out_hbm.at[pl.ds(eb * 8, CH * 8)])
        return carry

    lax.fori_loop(0, NCH, chunk, 0)


@functools.cache
def _make_scA():
    mesh = plsc.VectorSubcoreMesh(core_axis_name="c", subcore_axis_name="s")
    return pl.kernel(
        _scA_body,
        out_type=jax.ShapeDtypeStruct((E * 8,), jnp.float32),
        mesh=mesh,
        compiler_params=pltpu.CompilerParams(needs_layout_passes=False),
        scratch_types=[
            pltpu.VMEM((N * 3,), jnp.float32),
            pltpu.VMEM((N * 3,), jnp.float32),
            pltpu.VMEM((CH,), jnp.int32),
            pltpu.VMEM((CH,), jnp.int32),
            pltpu.VMEM((CH * 8,), jnp.float32),
        ],
    )


def _gelu(x):
    return 0.5 * x * (1.0 + lax.erf(x * 0.7071067811865476))


def _mlp_body(agg_ref, w1_ref, b1_ref, w2_ref, b2_ref, w3_ref, b3_ref, out_ref):
    a = agg_ref[...]
    h = jnp.dot(a, w1_ref[...], preferred_element_type=jnp.float32) + b1_ref[...]
    h = _gelu(h)
    h = jnp.dot(h, w2_ref[...], preferred_element_type=jnp.float32) + b2_ref[...]
    h = _gelu(h)
    out_ref[...] = jnp.dot(h, w3_ref[...], preferred_element_type=jnp.float32) + b3_ref[...]


def _mlp(agg8, W1p, b1, W2, b2, W3, b3):
    return pl.pallas_call(
        _mlp_body,
        grid=(N_BLOCKS,),
        in_specs=[
            pl.BlockSpec((EB, 8), lambda i: (i, 0)),
            pl.BlockSpec((8, 256), lambda i: (0, 0)),
            pl.BlockSpec((1, 256), lambda i: (0, 0)),
            pl.BlockSpec((256, 256), lambda i: (0, 0)),
            pl.BlockSpec((1, 256), lambda i: (0, 0)),
            pl.BlockSpec((256, 128), lambda i: (0, 0)),
            pl.BlockSpec((1, 128), lambda i: (0, 0)),
        ],
        out_specs=pl.BlockSpec((EB, F), lambda i: (i, 0)),
        out_shape=jax.ShapeDtypeStruct((E, F), jnp.float32),
    )(agg8, W1p, b1, W2, b2, W3, b3)


def _scC_body(kern_hbm, fy_hbm, src_hbm, qry_hbm, psums_hbm, pcnt_hbm,
              src_v, qry_v, fy_v, k_v, cnt_v, sums_sh, sem):
    cid = lax.axis_index("c")
    sid = lax.axis_index("s")
    wid = sid * NC + cid
    base = wid * EPW

    def zc(j, c):
        cnt_v[pl.ds(j * L, L)] = jnp.zeros((L,), jnp.float32)
        return c

    lax.fori_loop(0, N // L, zc, 0)

    def zz(r, c):
        for k in range(F // L):
            k_v[r, pl.ds(k * L, L)] = jnp.zeros((L,), jnp.float32)
        return c

    lax.fori_loop(0, ZR, zz, 0)

    @pl.when(sid < NTO)
    def _():
        for j in range(RPT // ZR):
            pltpu.sync_copy(k_v.at[pl.ds(0, ZR)], sums_sh.at[pl.ds(sid * RPT + j * ZR, ZR)])

    plsc.subcore_barrier()

    ones = jnp.ones((L,), jnp.float32)

    def chunk(i, carry):
        eb = base + i * CH
        pltpu.sync_copy(src_hbm.at[pl.ds(eb, CH)], src_v)
        pltpu.sync_copy(qry_hbm.at[pl.ds(eb, CH)], qry_v)
        cp = pltpu.async_copy(fy_hbm.at[src_v], fy_v, sem)
        pltpu.sync_copy(kern_hbm.at[pl.ds(eb, CH)], k_v)
        cp.wait()

        def mulrow(r, c2):
            for k in range(F // L):
                k_v[r, pl.ds(k * L, L)] = k_v[r, pl.ds(k * L, L)] * fy_v[r, pl.ds(k * L, L)]
            return c2

        lax.fori_loop(0, CH, mulrow, 0)
        for v in range(CH // L):
            plsc.addupdate_scatter(cnt_v, [qry_v[pl.ds(v * L, L)]], ones)
        pltpu.sync_copy(k_v, sums_sh.at[qry_v], add=True)
        return carry

    lax.fori_loop(0, NCH, chunk, 0)
    plsc.subcore_barrier()

    @pl.when(sid < NTO)
    def _():
        for j in range(RPT // WR):
            r0 = sid * RPT + j * WR
            pltpu.sync_copy(sums_sh.at[pl.ds(r0, WR)], psums_hbm.at[cid, pl.ds(r0, WR)])

    pltpu.sync_copy(cnt_v, pcnt_hbm.at[pl.ds(wid * N, N)])


@functools.cache
def _make_scC():
    mesh = plsc.VectorSubcoreMesh(core_axis_name="c", subcore_axis_name="s")
    return pl.kernel(
        _scC_body,
        out_type=[
            jax.ShapeDtypeStruct((NC, N, F), jnp.float32),
            jax.ShapeDtypeStruct((NW * N,), jnp.float32),
        ],
        mesh=mesh,
        compiler_params=pltpu.CompilerParams(needs_layout_passes=False),
        scratch_types=[
            pltpu.VMEM((CH,), jnp.int32),
            pltpu.VMEM((CH,), jnp.int32),
            pltpu.VMEM((CH, F), jnp.float32),
            pltpu.VMEM((CH, F), jnp.float32),
            pltpu.VMEM((N,), jnp.float32),
            pltpu.VMEM_SHARED((N, F), jnp.float32),
            pltpu.SemaphoreType.DMA,
        ],
    )


def _comb_body(ps_ref, pc_ref, out_ref):
    s = ps_ref[0, :, :] + ps_ref[1, :, :]
    c = jnp.sum(pc_ref[...], axis=0)
    out_ref[...] = s / jnp.clip(c, 1.0, None)[:, None]


def _combine(psums, pcnt):
    return pl.pallas_call(
        _comb_body,
        out_shape=jax.ShapeDtypeStruct((N, F), jnp.float32),
    )(psums, pcnt)


def kernel(y_pos, x_pos, edge_index, f_y, W1, b1, W2, b2, W3, b3):
    qry = edge_index[0]
    src = edge_index[1]
    agg_flat = _make_scA()(y_pos.reshape(-1), x_pos.reshape(-1), src, qry)
    agg8 = agg_flat.reshape(E, 8)
    W1p = jnp.pad(W1, ((0, 2), (0, 0)))
    kern = _mlp(agg8, W1p, b1[None, :], W2, b2[None, :], W3, b3[None, :])
    psums, pcnt = _make_scC()(kern, f_y, src, qry)
    return _combine(psums, pcnt.reshape(NW, N))


# speedup vs baseline: 7.9100x; 1.5096x over previous
import functools

import jax
import jax.numpy as jnp
from jax import lax
from jax.experimental import pallas as pl
from jax.experimental.pallas import tpu as pltpu
from jax.experimental.pallas import tpu_sc as plsc

E = 320000
N = 10000
F = 128
NC = 2
NS = 16
L = 16
NW = NC * NS
CH = 80
EPW = E // NW
NCH = EPW // CH
SECS = 5
SCH = NCH // SECS
PAIRS = (SCH - 1) // 2

NTO = 10
RPT = N // NTO
ZR = 40
WR = 200

EB = 1280
N_BLOCKS = E // EB


def _scA_body(y_hbm, x_hbm, src3_hbm, qry3_hbm, out_hbm, pcnt_hbm,
              y_v, x_v, srcs, qrys, o_v0, o_v1, cnt_v, so):
    o_bufs = (o_v0, o_v1)
    wid = lax.axis_index("s") * NC + lax.axis_index("c")
    base = wid * EPW
    pltpu.sync_copy(y_hbm, y_v)
    pltpu.sync_copy(x_hbm, x_v)
    pltpu.sync_copy(src3_hbm.at[wid], srcs)
    pltpu.sync_copy(qry3_hbm.at[wid], qrys)

    def zero(j, c):
        for b in range(2):
            o_bufs[b][pl.ds(j * L, L)] = jnp.zeros((L,), jnp.float32)
        return c

    lax.fori_loop(0, CH * 8 // L, zero, 0)

    def zc(j, c):
        cnt_v[pl.ds(j * L, L)] = jnp.zeros((L,), jnp.float32)
        return c

    lax.fori_loop(0, N // L, zc, 0)

    rows0 = lax.iota(jnp.int32, L)
    ones = jnp.ones((L,), jnp.float32)

    def fill(b, c):
        for v in range(CH // L):
            s3 = srcs[c, pl.ds(v * L, L)] * 3
            q = qrys[c, pl.ds(v * L, L)]
            o8 = (rows0 + v * L) * 8
            for k in range(3):
                plsc.store_scatter(o_bufs[b], [o8 + k], plsc.load_gather(y_v, [s3 + k]))
                plsc.store_scatter(o_bufs[b], [o8 + 3 + k], plsc.load_gather(x_v, [q * 3 + k]))
            plsc.addupdate_scatter(cnt_v, [q], ones)

    def st_desc(b, c):
        return pltpu.make_async_copy(
            o_bufs[b], out_hbm.at[pl.ds((base + c * CH) * 8, CH * 8)], so.at[b])

    def pair(j, carry):
        c0 = 2 * j
        c1 = c0 + 1

        @pl.when(j > 0)
        def _():
            st_desc(0, c0).wait()

        fill(0, c0)
        st_desc(0, c0).start()

        @pl.when(j > 0)
        def _():
            st_desc(1, c1).wait()

        fill(1, c1)
        st_desc(1, c1).start()
        return carry

    lax.fori_loop(0, NCH // 2, pair, 0)
    st_desc(0, NCH - 1).wait()
    fill(0, NCH - 1)
    st_desc(0, NCH - 1).start()
    st_desc(0, NCH - 1).wait()
    st_desc(1, NCH - 2).wait()
    pltpu.sync_copy(cnt_v, pcnt_hbm.at[pl.ds(wid * N, N)])


@functools.cache
def _make_scA():
    mesh = plsc.VectorSubcoreMesh(core_axis_name="c", subcore_axis_name="s")
    return pl.kernel(
        _scA_body,
        out_type=[
            jax.ShapeDtypeStruct((E * 8,), jnp.float32),
            jax.ShapeDtypeStruct((NW * N,), jnp.float32),
        ],
        mesh=mesh,
        compiler_params=pltpu.CompilerParams(needs_layout_passes=False),
        scratch_types=[
            pltpu.VMEM((N * 3,), jnp.float32),
            pltpu.VMEM((N * 3,), jnp.float32),
            pltpu.VMEM((NCH, CH), jnp.int32),
            pltpu.VMEM((NCH, CH), jnp.int32),
            pltpu.VMEM((CH * 8,), jnp.float32),
            pltpu.VMEM((CH * 8,), jnp.float32),
            pltpu.VMEM((N,), jnp.float32),
            pltpu.SemaphoreType.DMA((2,)),
        ],
    )


def _gelu2(x):
    return x * (1.0 + lax.erf(x * 0.7071067811865476))


def _mlp_body(agg_ref, w1_ref, b1_ref, w2_ref, b2_ref, w3_ref, b3_ref, out_ref):
    a = agg_ref[...]
    h = jnp.dot(a, w1_ref[...], preferred_element_type=jnp.float32) + b1_ref[...]
    h = _gelu2(h)
    h = jnp.dot(h, w2_ref[...], preferred_element_type=jnp.float32) + b2_ref[...]
    h = _gelu2(h)
    out_ref[...] = jnp.dot(h, w3_ref[...], preferred_element_type=jnp.float32) + b3_ref[...]


def _mlp(agg8, W1p, b1, W2, b2, W3, b3):
    return pl.pallas_call(
        _mlp_body,
        grid=(N_BLOCKS,),
        in_specs=[
            pl.BlockSpec((EB, 8), lambda i: (i, 0)),
            pl.BlockSpec((8, 256), lambda i: (0, 0)),
            pl.BlockSpec((1, 256), lambda i: (0, 0)),
            pl.BlockSpec((256, 256), lambda i: (0, 0)),
            pl.BlockSpec((1, 256), lambda i: (0, 0)),
            pl.BlockSpec((256, 128), lambda i: (0, 0)),
            pl.BlockSpec((1, 128), lambda i: (0, 0)),
        ],
        out_specs=pl.BlockSpec((EB, F), lambda i: (i, 0)),
        out_shape=jax.ShapeDtypeStruct((E, F), jnp.float32),
    )(agg8, W1p, b1, W2, b2, W3, b3)


def _scC_body(kern_hbm, fy_hbm, src4_hbm, qry4_hbm, psums_hbm,
              srcs, qrys, sidx0, sidx1, qidx0, qidx1,
              fy_v0, fy_v1, k_v0, k_v1, sums_sh, sfy, sk, ss):
    sidx = (sidx0, sidx1)
    qidx = (qidx0, qidx1)
    fy_b = (fy_v0, fy_v1)
    k_b = (k_v0, k_v1)
    cid = lax.axis_index("c")
    sid = lax.axis_index("s")
    wid = sid * NC + cid
    base = wid * EPW

    def zz(r, c):
        for k in range(F // L):
            k_v0[r, pl.ds(k * L, L)] = jnp.zeros((L,), jnp.float32)
        return c

    lax.fori_loop(0, ZR, zz, 0)

    @pl.when(sid < NTO)
    def _():
        for j in range(RPT // ZR):
            pltpu.sync_copy(k_v0.at[pl.ds(0, ZR)],
                            sums_sh.at[pl.ds(sid * RPT + j * ZR, ZR)])

    plsc.subcore_barrier()

    def fy_desc(b):
        return pltpu.make_async_copy(fy_hbm.at[sidx[b]], fy_b[b], sfy.at[b])

    def k_desc(b, g):
        return pltpu.make_async_copy(
            kern_hbm.at[pl.ds(base + g * CH, CH)], k_b[b], sk.at[b])

    def s_desc(b):
        return pltpu.make_async_copy(k_b[b], sums_sh.at[qidx[b]], ss.at[b])

    def issue(b, c, g):
        for v in range(CH // L):
            sidx[b][pl.ds(v * L, L)] = srcs[c, pl.ds(v * L, L)]
            qidx[b][pl.ds(v * L, L)] = qrys[c, pl.ds(v * L, L)]
        fy_desc(b).start()
        k_desc(b, g).start()

    def process(b):
        fy_desc(b).wait()
        k_desc(b, 0).wait()

        def mulrow(r, c2):
            for k in range(F // L):
                k_b[b][r, pl.ds(k * L, L)] = (
                    k_b[b][r, pl.ds(k * L, L)] * fy_b[b][r, pl.ds(k * L, L)])
            return c2

        lax.fori_loop(0, CH, mulrow, 0)
        pltpu.async_copy(k_b[b], sums_sh.at[qidx[b]], ss.at[b], add=True)

    def section(sec, carry):
        g0 = sec * SCH
        pltpu.sync_copy(src4_hbm.at[wid, sec], srcs)
        pltpu.sync_copy(qry4_hbm.at[wid, sec], qrys)

        issue(0, 0, g0)

        def pair(j, carry2):
            l0 = 2 * j
            l1 = l0 + 1

            @pl.when(j > 0)
            def _():
                s_desc(1).wait()

            issue(1, l1, g0 + l1)
            process(0)
            s_desc(0).wait()
            issue(0, l0 + 2, g0 + l0 + 2)
            process(1)
            return carry2

        lax.fori_loop(0, PAIRS, pair, 0)
        process(0)
        s_desc(0).wait()
        s_desc(1).wait()
        return carry

    lax.fori_loop(0, SECS, section, 0)
    plsc.subcore_barrier()

    @pl.when(sid < NTO)
    def _():
        for j in range(RPT // WR):
            r0 = sid * RPT + j * WR
            pltpu.sync_copy(sums_sh.at[pl.ds(r0, WR)], psums_hbm.at[cid, pl.ds(r0, WR)])


@functools.cache
def _make_scC():
    mesh = plsc.VectorSubcoreMesh(core_axis_name="c", subcore_axis_name="s")
    return pl.kernel(
        _scC_body,
        out_type=jax.ShapeDtypeStruct((NC, N, F), jnp.float32),
        mesh=mesh,
        compiler_params=pltpu.CompilerParams(needs_layout_passes=False),
        scratch_types=[
            pltpu.VMEM((SCH, CH), jnp.int32),
            pltpu.VMEM((SCH, CH), jnp.int32),
            pltpu.VMEM((CH,), jnp.int32),
            pltpu.VMEM((CH,), jnp.int32),
            pltpu.VMEM((CH,), jnp.int32),
            pltpu.VMEM((CH,), jnp.int32),
            pltpu.VMEM((CH, F), jnp.float32),
            pltpu.VMEM((CH, F), jnp.float32),
            pltpu.VMEM((CH, F), jnp.float32),
            pltpu.VMEM((CH, F), jnp.float32),
            pltpu.VMEM_SHARED((N, F), jnp.float32),
            pltpu.SemaphoreType.DMA((2,)),
            pltpu.SemaphoreType.DMA((2,)),
            pltpu.SemaphoreType.DMA((2,)),
        ],
    )


def _comb_body(ps_ref, pc_ref, out_ref):
    c = jnp.clip(jnp.sum(pc_ref[...], axis=0), 1.0, None)[:, None]
    out_ref[...] = (ps_ref[0, :, :] + ps_ref[1, :, :]) / c


def _combine(psums, pcnt):
    return pl.pallas_call(
        _comb_body,
        out_shape=jax.ShapeDtypeStruct((N, F), jnp.float32),
    )(psums, pcnt)


def kernel(y_pos, x_pos, edge_index, f_y, W1, b1, W2, b2, W3, b3):
    qry = edge_index[0]
    src = edge_index[1]
    src3 = src.reshape(NW, NCH, CH)
    qry3 = qry.reshape(NW, NCH, CH)
    agg_flat, pcnt = _make_scA()(y_pos.reshape(-1), x_pos.reshape(-1), src3, qry3)
    agg8 = agg_flat.reshape(E, 8)
    W1p = jnp.pad(W1, ((0, 2), (0, 0)))
    kern = _mlp(agg8, W1p, b1[None, :], 0.5 * W2, b2[None, :], 0.5 * W3, b3[None, :])
    src4 = src.reshape(NW, SECS, SCH, CH)
    qry4 = qry.reshape(NW, SECS, SCH, CH)
    psums = _make_scC()(kern, f_y, src4, qry4)
    return _combine(psums, pcnt.reshape(NW, N))
